# speedup vs baseline: 1.2310x; 1.2310x over previous
import functools

import jax
import jax.numpy as jnp
from jax import lax
from jax.experimental import pallas as pl
from jax.experimental.pallas import tpu as pltpu
from jax.experimental.pallas import tpu_sc as plsc

N, D, H, NB = 10000, 128, 128, 32
DEPTH = 6

_info = plsc.get_sparse_core_info()
NC, NS, L = _info.num_cores, _info.num_subcores, _info.num_lanes
NW = NC * NS
CHUNK = 320
NPAD = NW * CHUNK
G = 80
C = CHUNK // G


def _sc_gather_sum(idx_r, msg):
    mesh = plsc.VectorSubcoreMesh(core_axis_name="c", subcore_axis_name="s")

    @functools.partial(
        pl.kernel,
        out_type=jax.ShapeDtypeStruct((NPAD, D), jnp.float32),
        mesh=mesh,
        scratch_types=[
            pltpu.VMEM((NB, C, G), jnp.int32),
            pltpu.VMEM((CHUNK, D), jnp.float32),
            pltpu.VMEM((CHUNK, D), jnp.float32),
            pltpu.SemaphoreType.DMA,
        ],
    )
    def body(idx_hbm, msg_hbm, out_hbm, idx_v, buf_v, acc_v, sem):
        wid = lax.axis_index("s") * NC + lax.axis_index("c")
        base = wid * CHUNK
        pltpu.sync_copy(idx_hbm.at[wid], idx_v)

        def gather_cols(j, dst):
            cps = [
                pltpu.async_copy(
                    msg_hbm.at[idx_v.at[j, c]], dst.at[pl.ds(c * G, G)], sem
                )
                for c in range(C)
            ]
            for cp in cps:
                cp.wait()

        gather_cols(0, acc_v)

        def add_rows(i, _):
            for l in range(D // L):
                sl = pl.ds(l * L, L)
                acc_v[i, sl] += buf_v[i, sl]
            return 0

        def col_step(j, _):
            gather_cols(j, buf_v)
            lax.fori_loop(0, CHUNK, add_rows, 0)
            return 0

        lax.fori_loop(1, NB, col_step, 0)
        pltpu.sync_copy(acc_v, out_hbm.at[pl.ds(base, CHUNK)])

    return body(idx_r, msg)


def _tc_matmul_relu(x, wt):
    bm = 1024

    def body(x_ref, w_ref, o_ref):
        o_ref[...] = jnp.maximum(
            jnp.dot(x_ref[...], w_ref[...], preferred_element_type=jnp.float32),
            0.0,
        )

    return pl.pallas_call(
        body,
        grid=(NPAD // bm,),
        in_specs=[
            pl.BlockSpec((bm, D), lambda i: (i, 0)),
            pl.BlockSpec((D, H), lambda i: (0, 0)),
        ],
        out_specs=pl.BlockSpec((bm, H), lambda i: (i, 0)),
        out_shape=jax.ShapeDtypeStruct((NPAD, H), jnp.float32),
    )(x, wt)


def kernel(init_messages, init_attached_features, a2nei, a2attached, W_i, W_h):
    del init_attached_features, a2attached
    idx = jnp.pad(a2nei.astype(jnp.int32), ((0, NPAD - N), (0, 0)))
    idx_r = (
        idx.T.reshape(NB, NW, C, G).transpose(1, 0, 2, 3)
    )

    x = jnp.pad(init_messages, ((0, NPAD - N), (0, 0)))
    msg = _tc_matmul_relu(x, W_i.T)
    for _ in range(DEPTH - 1):
        s = _sc_gather_sum(idx_r, msg)
        msg = _tc_matmul_relu(s, W_h.T)
    return msg[:N]


# speedup vs baseline: 1.4228x; 1.1558x over previous
import functools

import jax
import jax.numpy as jnp
from jax import lax
from jax.experimental import pallas as pl
from jax.experimental.pallas import tpu as pltpu
from jax.experimental.pallas import tpu_sc as plsc

N, D, H, NB = 10000, 128, 128, 32
DEPTH = 6

_info = plsc.get_sparse_core_info()
NC, NS, L = _info.num_cores, _info.num_subcores, _info.num_lanes
NW = NC * NS
CHUNK = 320
NPAD = NW * CHUNK
G = 80
C = CHUNK // G


def _sc_gather_sum(idx_r, msg):
    mesh = plsc.VectorSubcoreMesh(core_axis_name="c", subcore_axis_name="s")

    @functools.partial(
        pl.kernel,
        out_type=jax.ShapeDtypeStruct((NPAD, D), jnp.float32),
        mesh=mesh,
        scratch_types=[
            pltpu.VMEM((NB, C, G), jnp.int32),
            pltpu.VMEM((CHUNK, D), jnp.float32),
            pltpu.VMEM((CHUNK, D), jnp.float32),
            [pltpu.SemaphoreType.DMA] * C,
        ],
    )
    def body(idx_hbm, msg_hbm, out_hbm, idx_v, buf_v, acc_v, sems):
        wid = lax.axis_index("s") * NC + lax.axis_index("c")
        base = wid * CHUNK
        pltpu.sync_copy(idx_hbm.at[wid], idx_v)

        def start(j, c):
            pltpu.async_copy(
                msg_hbm.at[idx_v.at[j, c]], buf_v.at[pl.ds(c * G, G)], sems[c]
            )

        def wait(c):
            pltpu.make_async_copy(
                msg_hbm.at[idx_v.at[0, c]], buf_v.at[pl.ds(c * G, G)], sems[c]
            ).wait()

        def zero_rows(i, _):
            z = jnp.zeros((L,), jnp.float32)
            for l in range(D // L):
                acc_v[i, pl.ds(l * L, L)] = z
            return 0

        lax.fori_loop(0, CHUNK, zero_rows, 0)

        def accum_chunk(c):
            def add_rows(i, _):
                r = c * G + i
                for l in range(D // L):
                    sl = pl.ds(l * L, L)
                    plsc.addupdate(acc_v.at[r, sl], buf_v[r, sl])
                return 0

            lax.fori_loop(0, G, add_rows, 0)

        for c in range(C):
            start(0, c)

        def col_step(j, _):
            for c in range(C):
                wait(c)
                accum_chunk(c)
                start(j + 1, c)
            return 0

        lax.fori_loop(0, NB - 1, col_step, 0)
        for c in range(C):
            wait(c)
            accum_chunk(c)
        pltpu.sync_copy(acc_v, out_hbm.at[pl.ds(base, CHUNK)])

    return body(idx_r, msg)


def _tc_matmul_relu(x, wt):
    bm = 1024

    def body(x_ref, w_ref, o_ref):
        o_ref[...] = jnp.maximum(
            jnp.dot(x_ref[...], w_ref[...], preferred_element_type=jnp.float32),
            0.0,
        )

    return pl.pallas_call(
        body,
        grid=(NPAD // bm,),
        in_specs=[
            pl.BlockSpec((bm, D), lambda i: (i, 0)),
            pl.BlockSpec((D, H), lambda i: (0, 0)),
        ],
        out_specs=pl.BlockSpec((bm, H), lambda i: (i, 0)),
        out_shape=jax.ShapeDtypeStruct((NPAD, H), jnp.float32),
    )(x, wt)


def kernel(init_messages, init_attached_features, a2nei, a2attached, W_i, W_h):
    del init_attached_features, a2attached
    idx = jnp.pad(a2nei.astype(jnp.int32), ((0, NPAD - N), (0, 0)))
    idx_r = (
        idx.T.reshape(NB, NW, C, G).transpose(1, 0, 2, 3)
    )

    x = jnp.pad(init_messages, ((0, NPAD - N), (0, 0)))
    msg = _tc_matmul_relu(x, W_i.T)
    for _ in range(DEPTH - 1):
        s = _sc_gather_sum(idx_r, msg)
        msg = _tc_matmul_relu(s, W_h.T)
    return msg[:N]
